# baseline (speedup 1.0000x reference)
import functools

import jax
import jax.numpy as jnp
from jax import lax
from jax.experimental import pallas as pl
from jax.experimental.pallas import tpu as pltpu
from jax.experimental.pallas import tpu_sc as plsc

VOCAB = 1000000
MAXLEN = 200
EMBED = 64
BATCH = 4096

NUM_CORES = 2
NUM_SUBCORES = 16
NW = NUM_CORES * NUM_SUBCORES
TOTAL = BATCH * MAXLEN
PER_W = TOTAL // NW
CHUNK = 128
NCHUNK = PER_W // CHUNK
NBUF = 4
PREFETCH = 2
LANES = 16
VPR = EMBED // LANES


_mesh = plsc.VectorSubcoreMesh(core_axis_name="c", subcore_axis_name="s")


@functools.partial(
    pl.kernel,
    out_type=jax.ShapeDtypeStruct((TOTAL, EMBED), jnp.float32),
    mesh=_mesh,
    scratch_types=[
        pltpu.VMEM((PER_W,), jnp.int32),
        pltpu.VMEM((MAXLEN, EMBED), jnp.float32),
        pltpu.VMEM((NBUF, CHUNK, EMBED), jnp.float32),
        pltpu.SemaphoreType.DMA((NBUF,)),
        pltpu.SemaphoreType.DMA((NBUF,)),
    ],
    compiler_params=pltpu.CompilerParams(use_tc_tiling_on_sc=False),
)
def _embed_sc(x_hbm, tok_hbm, pos_hbm, out_hbm, idx_v, pos_v, bufs, gsem, ssem):
    wid = lax.axis_index("s") * NUM_CORES + lax.axis_index("c")
    base = wid * PER_W

    pltpu.sync_copy(pos_hbm, pos_v)
    pltpu.sync_copy(x_hbm.at[pl.ds(base, PER_W)], idx_v)

    def start_gather(g, b):
        pltpu.async_copy(
            tok_hbm.at[idx_v.at[pl.ds(g * CHUNK, CHUNK)]], bufs.at[b], gsem.at[b])

    def wait_gather(g, b):
        pltpu.make_async_copy(
            tok_hbm.at[idx_v.at[pl.ds(g * CHUNK, CHUNK)]], bufs.at[b], gsem.at[b]).wait()

    def start_scatter(g, b):
        pltpu.async_copy(
            bufs.at[b], out_hbm.at[pl.ds(base + g * CHUNK, CHUNK)], ssem.at[b])

    def wait_scatter(g, b):
        pltpu.make_async_copy(
            bufs.at[b], out_hbm.at[pl.ds(base + g * CHUNK, CHUNK)], ssem.at[b]).wait()

    def add_pos(g, b):
        pos0 = lax.rem(g * CHUNK, MAXLEN)

        @pl.loop(0, CHUNK)
        def _(r):
            p = pos0 + r
            p = jnp.where(p >= MAXLEN, p - MAXLEN, p)
            for v in range(VPR):
                plsc.addupdate(
                    bufs.at[b, r, pl.ds(v * LANES, LANES)],
                    pos_v[p, pl.ds(v * LANES, LANES)])

    for b in range(PREFETCH):
        start_gather(b, b)

    @pl.loop(0, NCHUNK // NBUF)
    def _(i):
        for b in range(NBUF):
            g = i * NBUF + b
            h = g + PREFETCH
            bh = (b + PREFETCH) % NBUF

            @pl.when(h < NCHUNK)
            def _():
                @pl.when(h >= NBUF)
                def _():
                    wait_scatter(h - NBUF, bh)

                start_gather(h, bh)

            wait_gather(g, b)
            add_pos(g, b)
            start_scatter(g, b)

    for b in range(NBUF):
        wait_scatter(NCHUNK - NBUF + b, b)


def kernel(x, token_table, pos_table):
    x_flat = x.reshape(TOTAL).astype(jnp.int32)
    out = _embed_sc(x_flat, token_table, pos_table)
    return out.reshape(BATCH, MAXLEN, EMBED)


# speedup vs baseline: 1.4705x; 1.4705x over previous
import functools

import jax
import jax.numpy as jnp
from jax import lax
from jax.experimental import pallas as pl
from jax.experimental.pallas import tpu as pltpu
from jax.experimental.pallas import tpu_sc as plsc

VOCAB = 1000000
MAXLEN = 200
EMBED = 64
BATCH = 4096

NUM_CORES = 2
NUM_SUBCORES = 16
NW = NUM_CORES * NUM_SUBCORES
TOTAL = BATCH * MAXLEN
PER_W = TOTAL // NW
CHUNK = 128
NCHUNK = PER_W // CHUNK
NBUF = 4
PREFETCH = 2
LANES = 16
VPR = EMBED // LANES


_mesh = plsc.VectorSubcoreMesh(core_axis_name="c", subcore_axis_name="s")


@functools.partial(
    pl.kernel,
    out_type=jax.ShapeDtypeStruct((TOTAL, EMBED), jnp.float32),
    mesh=_mesh,
    scratch_types=[
        pltpu.VMEM((PER_W,), jnp.int32),
        pltpu.VMEM((MAXLEN, EMBED), jnp.float32),
        pltpu.VMEM((NBUF, CHUNK, EMBED), jnp.float32),
        pltpu.SemaphoreType.DMA((NBUF,)),
        pltpu.SemaphoreType.DMA((NBUF,)),
    ],
    compiler_params=pltpu.CompilerParams(use_tc_tiling_on_sc=True),
)
def _embed_sc(x_hbm, tok_hbm, pos_hbm, out_hbm, idx_v, pos_v, bufs, gsem, ssem):
    wid = lax.axis_index("s") * NUM_CORES + lax.axis_index("c")
    base = wid * PER_W

    pltpu.sync_copy(pos_hbm, pos_v)
    pltpu.sync_copy(x_hbm.at[pl.ds(base, PER_W)], idx_v)

    def start_gather(g, b):
        @pl.loop(0, CHUNK // LANES)
        def _(u):
            tv = idx_v[pl.ds(g * CHUNK + u * LANES, LANES)]
            for j in range(LANES):
                tok = tv[j]
                pltpu.async_copy(tok_hbm.at[pl.ds(tok, 1), :],
                                 bufs.at[b, pl.ds(u * LANES + j, 1), :],
                                 gsem.at[b])

    def wait_gather(b):
        pltpu.make_async_copy(tok_hbm.at[pl.ds(0, CHUNK), :],
                              bufs.at[b], gsem.at[b]).wait()

    def start_scatter(g, b):
        pltpu.async_copy(
            bufs.at[b], out_hbm.at[pl.ds(base + g * CHUNK, CHUNK), :], ssem.at[b])

    def wait_scatter(g, b):
        pltpu.make_async_copy(
            bufs.at[b], out_hbm.at[pl.ds(base + g * CHUNK, CHUNK), :],
            ssem.at[b]).wait()

    def add_pos(g, b):
        pos0 = lax.rem(g * CHUNK, MAXLEN)

        @pl.loop(0, CHUNK)
        def _(r):
            p = pos0 + r
            p = jnp.where(p >= MAXLEN, p - MAXLEN, p)
            for v in range(VPR):
                plsc.addupdate(
                    bufs.at[b, r, pl.ds(v * LANES, LANES)],
                    pos_v[p, pl.ds(v * LANES, LANES)])

    for b in range(PREFETCH):
        start_gather(b, b)

    @pl.loop(0, NCHUNK // NBUF)
    def _(i):
        for b in range(NBUF):
            g = i * NBUF + b
            h = g + PREFETCH
            bh = (b + PREFETCH) % NBUF

            @pl.when(h < NCHUNK)
            def _():
                @pl.when(h >= NBUF)
                def _():
                    wait_scatter(h - NBUF, bh)

                start_gather(h, bh)

            wait_gather(b)
            add_pos(g, b)
            start_scatter(g, b)

    for b in range(NBUF):
        wait_scatter(NCHUNK - NBUF + b, b)


def kernel(x, token_table, pos_table):
    x_flat = x.reshape(TOTAL).astype(jnp.int32)
    out = _embed_sc(x_flat, token_table, pos_table)
    return out.reshape(BATCH, MAXLEN, EMBED)
